# Initial kernel scaffold; baseline (speedup 1.0000x reference)
import jax
import jax.numpy as jnp
from jax.experimental import pallas as pl


def kernel(node_features, edges, edge_features, params):
    raise NotImplementedError("write your pallas kernel here")


# speedup vs baseline: 1.9574x; 1.9574x over previous
import functools

import jax
import jax.numpy as jnp
from jax import lax
from jax.experimental import pallas as pl
from jax.experimental.pallas import tpu as pltpu
from jax.experimental.pallas import tpu_sc as plsc

NC = 2
NS = 16
LANE = 16
BE = 128


def _row_mesh():
    return plsc.VectorSubcoreMesh(core_axis_name="c", subcore_axis_name="s")


@functools.lru_cache(maxsize=None)
def _edge_pass_kernel(n_acc, e_pad):
    tpt = e_pad // NS
    nb = tpt // BE
    rows_t = n_acc // NS
    zr = rows_t // 8

    @functools.partial(
        pl.kernel,
        mesh=_row_mesh(),
        out_type=(jax.ShapeDtypeStruct((NC * n_acc, LANE), jnp.float32),
                  jax.ShapeDtypeStruct((NC * n_acc, LANE), jnp.float32)),
        compiler_params=pltpu.CompilerParams(use_tc_tiling_on_sc=False),
        scratch_types=[
            pltpu.VMEM((BE,), jnp.int32),
            pltpu.VMEM((BE,), jnp.int32),
            pltpu.VMEM((BE,), jnp.int32),
            pltpu.VMEM((BE, LANE), jnp.float32),
            pltpu.VMEM((BE, LANE), jnp.float32),
            pltpu.VMEM((BE, LANE), jnp.float32),
            pltpu.VMEM((BE, LANE), jnp.float32),
            pltpu.VMEM((BE, LANE), jnp.float32),
            pltpu.VMEM((zr, LANE), jnp.float32),
            pltpu.VMEM_SHARED((n_acc, LANE), jnp.float32),
            pltpu.SemaphoreType.DMA,
            pltpu.SemaphoreType.DMA,
        ],
    )
    def k(srcx2_h, dstx2_h, dst_h, a2_h, b2_h, c2_h, s_out_h, d_out_h,
          sidx, didx, bidx, arow, brow, crow, srow, vones, zbuf, acc,
          sem_a, sem_b):
        c = lax.axis_index("c")
        s = lax.axis_index("s")
        half_off = c * n_acc

        def zb(i, carry):
            zbuf[i, :] = jnp.zeros((LANE,), jnp.float32)
            return carry
        lax.fori_loop(0, zr, zb, 0)

        def ob(i, carry):
            vones[i, :] = jnp.ones((LANE,), jnp.float32)
            return carry
        lax.fori_loop(0, BE, ob, 0)

        row0 = s * rows_t
        tile_base = s * tpt

        def zero_acc():
            for q in range(rows_t // zr):
                pltpu.sync_copy(zbuf, acc.at[pl.ds(row0 + q * zr, zr)])

        zero_acc()
        plsc.subcore_barrier()

        def dbatch(j, carry):
            pltpu.sync_copy(dst_h.at[pl.ds(tile_base + j * BE, BE)], didx)
            pltpu.sync_copy(vones, acc.at[didx], add=True)
            return carry
        lax.fori_loop(0, nb, dbatch, 0)
        plsc.subcore_barrier()
        pltpu.sync_copy(acc.at[pl.ds(row0, rows_t)],
                        d_out_h.at[pl.ds(half_off + row0, rows_t)])
        plsc.subcore_barrier()

        zero_acc()
        plsc.subcore_barrier()

        def batch(j, carry):
            base = tile_base + j * BE
            pltpu.sync_copy(srcx2_h.at[pl.ds(c * e_pad + base, BE)], sidx)
            pltpu.sync_copy(dstx2_h.at[pl.ds(c * e_pad + base, BE)], bidx)
            pltpu.sync_copy(dst_h.at[pl.ds(base, BE)], didx)
            ga = pltpu.async_copy(a2_h.at[sidx], arow, sem_a)
            gb = pltpu.async_copy(b2_h.at[bidx], brow, sem_b)
            pltpu.sync_copy(c2_h.at[pl.ds(c * e_pad + base, BE)], crow)
            ga.wait()
            gb.wait()

            def comp(i, cc):
                srow[i, :] = jnp.maximum(arow[i, :] + brow[i, :] + crow[i, :],
                                         0.0)
                return cc
            lax.fori_loop(0, BE, comp, 0)
            pltpu.sync_copy(srow, acc.at[didx], add=True)
            return carry

        lax.fori_loop(0, nb, batch, 0)
        plsc.subcore_barrier()
        pltpu.sync_copy(acc.at[pl.ds(row0, rows_t)],
                        s_out_h.at[pl.ds(half_off + row0, rows_t)])

    return k


def _mm(a, b):
    return jax.lax.dot_general(a, b, (((1,), (0,)), ((), ())),
                               precision=jax.lax.Precision.HIGHEST)


_BN = 2000
_EBN = 6256


def _full(a):
    return pl.BlockSpec(a.shape, lambda i: tuple(0 for _ in a.shape))


def _split_halves(x):
    return jnp.stack([x[:, :LANE], x[:, LANE:]], axis=0)


def _tc_pre(x, ew1, eb1, ew2, eb2, w1a, w1b, *, n, n_acc):
    nb = n // _BN

    def body(x_r, ew1_r, eb1_r, ew2_r, eb2_r, w1a_r, w1b_r, h_r, a_r, b_r):
        h = _mm(jnp.maximum(_mm(x_r[...], ew1_r[...]) + eb1_r[...], 0.0),
                ew2_r[...]) + eb2_r[...]
        h_r[...] = h
        a_r[...] = _split_halves(_mm(h, w1a_r[...]))
        b_r[...] = _split_halves(_mm(h, w1b_r[...]))

    h, a2, b2 = pl.pallas_call(
        body,
        grid=(nb,),
        in_specs=[pl.BlockSpec((_BN, x.shape[1]), lambda i: (i, 0)),
                  _full(ew1), _full(eb1), _full(ew2), _full(eb2),
                  _full(w1a), _full(w1b)],
        out_specs=[pl.BlockSpec((_BN, 2 * LANE), lambda i: (i, 0)),
                   pl.BlockSpec((2, _BN, LANE), lambda i: (0, i, 0)),
                   pl.BlockSpec((2, _BN, LANE), lambda i: (0, i, 0))],
        out_shape=[jax.ShapeDtypeStruct((n, 2 * LANE), jnp.float32),
                   jax.ShapeDtypeStruct((2, n_acc, LANE), jnp.float32),
                   jax.ShapeDtypeStruct((2, n_acc, LANE), jnp.float32)],
    )(x, ew1, eb1, ew2, eb2, w1a, w1b)
    return h, a2, b2


def _tc_c(ef_p, w1c0, b10, w1c1, b11, *, e_pad):
    nb = e_pad // _EBN

    def body(ef_r, w0_r, b0_r, w1_r, b1_r, c0_r, c1_r):
        e = ef_r[...]
        c0_r[...] = _split_halves(_mm(e, w0_r[...]) + b0_r[...])
        c1_r[...] = _split_halves(_mm(e, w1_r[...]) + b1_r[...])

    c0, c1 = pl.pallas_call(
        body,
        grid=(nb,),
        in_specs=[pl.BlockSpec((_EBN, ef_p.shape[1]), lambda i: (i, 0)),
                  _full(w1c0), _full(b10), _full(w1c1), _full(b11)],
        out_specs=[pl.BlockSpec((2, _EBN, LANE), lambda i: (0, i, 0)),
                   pl.BlockSpec((2, _EBN, LANE), lambda i: (0, i, 0))],
        out_shape=[jax.ShapeDtypeStruct((2, e_pad, LANE), jnp.float32),
                   jax.ShapeDtypeStruct((2, e_pad, LANE), jnp.float32)],
    )(ef_p, w1c0, b10, w1c1, b11)
    return c0, c1


def _tc_post(h, s2, d2, mw2, mb2, uw1, ub1, uw2, ub2, nxt, *, n, n_acc):
    nb = n // _BN
    have_next = nxt is not None
    nxt_ops = list(nxt) if have_next else []

    def body(h_r, sl_r, sh_r, d_r, mw2_r, mb2_r,
             uw1_r, ub1_r, uw2_r, ub2_r, *rest):
        deg = d_r[0, :, 0:1]
        agg = (_mm(sl_r[0], mw2_r[:LANE, :]) + _mm(sh_r[0], mw2_r[LANE:, :])
               + deg * mb2_r[...])
        hh = h_r[...]
        t = jnp.maximum(_mm(hh, uw1_r[:2 * LANE, :])
                        + _mm(agg, uw1_r[2 * LANE:, :]) + ub1_r[...], 0.0)
        hn = hh + _mm(t, uw2_r[...]) + ub2_r[...]
        if have_next:
            w1a_r, w1b_r, hn_ref, a_ref, b_ref = rest
            a_ref[...] = _split_halves(_mm(hn, w1a_r[...]))
            b_ref[...] = _split_halves(_mm(hn, w1b_r[...]))
        else:
            (hn_ref,) = rest
        hn_ref[...] = hn

    half_spec = [pl.BlockSpec((1, _BN, LANE), lambda i: (0, i, 0)),
                 pl.BlockSpec((1, _BN, LANE), lambda i: (1, i, 0))]
    in_specs = ([pl.BlockSpec((_BN, 2 * LANE), lambda i: (i, 0))]
                + half_spec
                + [pl.BlockSpec((1, _BN, LANE), lambda i: (0, i, 0))]
                + [_full(mw2), _full(mb2), _full(uw1), _full(ub1),
                   _full(uw2), _full(ub2)]
                + [_full(w) for w in nxt_ops])
    out_specs = [pl.BlockSpec((_BN, 2 * LANE), lambda i: (i, 0))]
    out_shape = [jax.ShapeDtypeStruct((n, 2 * LANE), jnp.float32)]
    if have_next:
        out_specs += [pl.BlockSpec((2, _BN, LANE), lambda i: (0, i, 0))] * 2
        out_shape += [jax.ShapeDtypeStruct((2, n_acc, LANE), jnp.float32)] * 2

    outs = pl.pallas_call(
        body,
        grid=(nb,),
        in_specs=in_specs,
        out_specs=out_specs,
        out_shape=out_shape,
    )(h, s2, s2, d2, mw2, mb2, uw1, ub1, uw2, ub2, *nxt_ops)
    return outs if have_next else (outs[0], None, None)


def _tc_head(h, hw1, hb1, hw2, hb2, *, n):
    nb = n // _BN

    def body(h_r, w1_r, b1_r, w2_r, b2_r, o_r):
        o_r[...] = _mm(jnp.maximum(_mm(h_r[...], w1_r[...]) + b1_r[...], 0.0),
                       w2_r[...]) + b2_r[...]

    out = pl.pallas_call(
        body,
        grid=(nb,),
        in_specs=[pl.BlockSpec((_BN, 2 * LANE), lambda i: (i, 0)),
                  _full(hw1), _full(hb1), _full(hw2), _full(hb2)],
        out_specs=pl.BlockSpec((_BN, 1), lambda i: (i, 0)),
        out_shape=jax.ShapeDtypeStruct((n, 1), jnp.float32),
    )(h, hw1, hb1, hw2, hb2)
    return out


def kernel(node_features, edges, edge_features, params):
    p = params
    n, f = node_features.shape
    e, de = edge_features.shape
    h_dim = p['enc_W1'].shape[1]
    assert h_dim == 2 * LANE

    tpt = -(-e // (NS * BE)) * BE
    e_pad = tpt * NS
    n_acc = -(-(n + 1) // (NS * 8)) * (NS * 8)

    pad = e_pad - e
    src_p = jnp.concatenate([edges[:, 0], jnp.zeros((pad,), edges.dtype)])
    dst_p = jnp.concatenate([edges[:, 1],
                             jnp.full((pad,), n, edges.dtype)])
    src_p = src_p.astype(jnp.int32)
    dst_p = dst_p.astype(jnp.int32)
    ef_p = jnp.concatenate([edge_features,
                            jnp.zeros((pad, de), jnp.float32)], axis=0)
    srcx2 = jnp.concatenate([src_p, src_p + n_acc])
    dstx2 = jnp.concatenate([dst_p, dst_p + n_acc])

    def rowvec(b):
        return b.reshape(1, -1)

    hh = 2 * LANE
    d2 = None

    h, a2, b2 = _tc_pre(node_features, p['enc_W1'], rowvec(p['enc_b1']),
                        p['enc_W2'], rowvec(p['enc_b2']),
                        p['msg0_W1'][:hh], p['msg0_W1'][hh:2 * hh],
                        n=n, n_acc=n_acc)
    c2_0, c2_1 = _tc_c(ef_p, p['msg0_W1'][2 * hh:], rowvec(p['msg0_b1']),
                       p['msg1_W1'][2 * hh:], rowvec(p['msg1_b1']),
                       e_pad=e_pad)

    rounds = 2
    for r in range(rounds):
        c2 = (c2_0, c2_1)[r]
        s2, d2r = _edge_pass_kernel(n_acc, e_pad)(
            srcx2, dstx2, dst_p,
            a2.reshape(NC * n_acc, LANE),
            b2.reshape(NC * n_acc, LANE),
            c2.reshape(NC * e_pad, LANE))
        s2 = s2.reshape(NC, n_acc, LANE)
        if d2 is None:
            d2 = d2r.reshape(NC, n_acc, LANE)
        nxt = None
        if r + 1 < rounds:
            w1n = p[f'msg{r + 1}_W1']
            nxt = (w1n[:hh], w1n[hh:2 * hh])
        h, a2, b2 = _tc_post(h, s2, d2,
                             p[f'msg{r}_W2'], rowvec(p[f'msg{r}_b2']),
                             p[f'upd{r}_W1'], rowvec(p[f'upd{r}_b1']),
                             p[f'upd{r}_W2'], rowvec(p[f'upd{r}_b2']),
                             nxt, n=n, n_acc=n_acc)

    out = _tc_head(h, p['head_W1'], rowvec(p['head_b1']),
                   p['head_W2'], rowvec(p['head_b2']), n=n)
    return out[:, 0]


# speedup vs baseline: 2.4763x; 1.2651x over previous
import functools

import jax
import jax.numpy as jnp
from jax import lax
from jax.experimental import pallas as pl
from jax.experimental.pallas import tpu as pltpu
from jax.experimental.pallas import tpu_sc as plsc

NC = 2
NS = 16
LANE = 16
BE = 128
SB = 4


def _row_mesh():
    return plsc.VectorSubcoreMesh(core_axis_name="c", subcore_axis_name="s")


@functools.lru_cache(maxsize=None)
def _edge_pass_kernel(n_acc, e_pad):
    tpt = e_pad // NS
    sbe = SB * BE
    ng = tpt // sbe
    rpt = tpt // BE
    rows_t = n_acc // NS

    @functools.partial(
        pl.kernel,
        mesh=_row_mesh(),
        out_type=(jax.ShapeDtypeStruct((NC * n_acc, LANE), jnp.float32),
                  jax.ShapeDtypeStruct((NC * n_acc, LANE), jnp.float32)),
        compiler_params=pltpu.CompilerParams(use_tc_tiling_on_sc=False),
        scratch_types=[
            pltpu.VMEM((SB, BE), jnp.int32),
            pltpu.VMEM((SB, BE), jnp.int32),
            pltpu.VMEM((SB, BE), jnp.int32),
            pltpu.VMEM((SB * BE, LANE), jnp.float32),
            pltpu.VMEM((SB * BE, LANE), jnp.float32),
            pltpu.VMEM((SB * BE, LANE), jnp.float32),
            pltpu.VMEM_SHARED((n_acc, LANE), jnp.float32),
            pltpu.SemaphoreType.DMA,
            pltpu.SemaphoreType.DMA,
            pltpu.SemaphoreType.DMA,
        ],
    )
    def k(srcx2_h, dstx2_h, dst2_h, a2_h, b2_h, c2_h, s_out_h, d_out_h,
          sidx, didx, bidx, arow, brow, srow, acc, sem_a, sem_b, sem_c):
        c = lax.axis_index("c")
        s = lax.axis_index("s")
        half_off = c * n_acc
        zr = SB * BE

        def fill(val, nrows):
            def fb(i, carry):
                srow[i, :] = jnp.full((LANE,), val, jnp.float32)
                return carry
            lax.fori_loop(0, nrows, fb, 0)

        row0 = s * rows_t
        tile_base = s * tpt
        tile_rbase = s * rpt

        def zero_acc():
            fill(0.0, zr)
            nz = rows_t // zr
            for q in range(nz):
                pltpu.sync_copy(srow.at[pl.ds(0, zr)],
                                acc.at[pl.ds(row0 + q * zr, zr)])
            rem = rows_t - nz * zr
            if rem:
                pltpu.sync_copy(srow.at[pl.ds(0, rem)],
                                acc.at[pl.ds(row0 + nz * zr, rem)])

        zero_acc()
        plsc.subcore_barrier()

        fill(1.0, BE)

        def dbatch(g, carry):
            pltpu.sync_copy(dst2_h.at[pl.ds(tile_rbase + g * SB, SB)], didx)
            for q in range(SB):
                pltpu.sync_copy(srow.at[pl.ds(0, BE)],
                                acc.at[didx.at[q]], add=True)
            return carry
        lax.fori_loop(0, ng, dbatch, 0)
        plsc.subcore_barrier()
        pltpu.sync_copy(acc.at[pl.ds(row0, rows_t)],
                        d_out_h.at[pl.ds(half_off + row0, rows_t)])
        plsc.subcore_barrier()

        zero_acc()
        plsc.subcore_barrier()

        def batch(g, carry):
            ebase = tile_base + g * sbe
            rbase = tile_rbase + g * SB
            cc = pltpu.async_copy(c2_h.at[pl.ds(c * e_pad + ebase, sbe)],
                                  srow, sem_c)
            pltpu.sync_copy(srcx2_h.at[pl.ds(c * rpt * NS + rbase, SB)], sidx)
            pltpu.sync_copy(dstx2_h.at[pl.ds(c * rpt * NS + rbase, SB)], bidx)
            pltpu.sync_copy(dst2_h.at[pl.ds(rbase, SB)], didx)
            gas = [pltpu.async_copy(a2_h.at[sidx.at[q]],
                                    arow.at[pl.ds(q * BE, BE)], sem_a)
                   for q in range(SB)]
            gbs = [pltpu.async_copy(b2_h.at[bidx.at[q]],
                                    brow.at[pl.ds(q * BE, BE)], sem_b)
                   for q in range(SB)]
            for d in gas:
                d.wait()
            for d in gbs:
                d.wait()
            cc.wait()

            def comp(i, carry2):
                srow[i, :] = jnp.maximum(arow[i, :] + brow[i, :] + srow[i, :],
                                         0.0)
                return carry2
            lax.fori_loop(0, sbe, comp, 0)
            for q in range(SB):
                pltpu.sync_copy(srow.at[pl.ds(q * BE, BE)],
                                acc.at[didx.at[q]], add=True)
            return carry

        lax.fori_loop(0, ng, batch, 0)
        plsc.subcore_barrier()
        pltpu.sync_copy(acc.at[pl.ds(row0, rows_t)],
                        s_out_h.at[pl.ds(half_off + row0, rows_t)])

    return k


def _mm(a, b):
    return jax.lax.dot_general(a, b, (((1,), (0,)), ((), ())),
                               precision=jax.lax.Precision.HIGHEST)


_BN = 2000


def _full(a):
    return pl.BlockSpec(a.shape, lambda i: tuple(0 for _ in a.shape))


def _split_halves(x):
    return jnp.stack([x[:, :LANE], x[:, LANE:]], axis=0)


def _tc_pre(x, ew1, eb1, ew2, eb2, w1a, w1b, *, n, n_acc):
    nb = n // _BN

    def body(x_r, ew1_r, eb1_r, ew2_r, eb2_r, w1a_r, w1b_r, h_r, a_r, b_r):
        h = _mm(jnp.maximum(_mm(x_r[...], ew1_r[...]) + eb1_r[...], 0.0),
                ew2_r[...]) + eb2_r[...]
        h_r[...] = h
        a_r[...] = _split_halves(_mm(h, w1a_r[...]))
        b_r[...] = _split_halves(_mm(h, w1b_r[...]))

    h, a2, b2 = pl.pallas_call(
        body,
        grid=(nb,),
        in_specs=[pl.BlockSpec((_BN, x.shape[1]), lambda i: (i, 0)),
                  _full(ew1), _full(eb1), _full(ew2), _full(eb2),
                  _full(w1a), _full(w1b)],
        out_specs=[pl.BlockSpec((_BN, 2 * LANE), lambda i: (i, 0)),
                   pl.BlockSpec((2, _BN, LANE), lambda i: (0, i, 0)),
                   pl.BlockSpec((2, _BN, LANE), lambda i: (0, i, 0))],
        out_shape=[jax.ShapeDtypeStruct((n, 2 * LANE), jnp.float32),
                   jax.ShapeDtypeStruct((2, n_acc, LANE), jnp.float32),
                   jax.ShapeDtypeStruct((2, n_acc, LANE), jnp.float32)],
    )(x, ew1, eb1, ew2, eb2, w1a, w1b)
    return h, a2, b2


def _tc_c(ef_p, w1c0, b10, w1c1, b11, *, e_pad):
    ebn = e_pad // 256
    nb = 256

    def body(ef_r, w0_r, b0_r, w1_r, b1_r, c0_r, c1_r):
        e = ef_r[...]
        c0_r[...] = _split_halves(_mm(e, w0_r[...]) + b0_r[...])
        c1_r[...] = _split_halves(_mm(e, w1_r[...]) + b1_r[...])

    c0, c1 = pl.pallas_call(
        body,
        grid=(nb,),
        in_specs=[pl.BlockSpec((ebn, ef_p.shape[1]), lambda i: (i, 0)),
                  _full(w1c0), _full(b10), _full(w1c1), _full(b11)],
        out_specs=[pl.BlockSpec((2, ebn, LANE), lambda i: (0, i, 0)),
                   pl.BlockSpec((2, ebn, LANE), lambda i: (0, i, 0))],
        out_shape=[jax.ShapeDtypeStruct((2, e_pad, LANE), jnp.float32),
                   jax.ShapeDtypeStruct((2, e_pad, LANE), jnp.float32)],
    )(ef_p, w1c0, b10, w1c1, b11)
    return c0, c1


def _tc_post(h, s2, d2, mw2, mb2, uw1, ub1, uw2, ub2, nxt, *, n, n_acc):
    nb = n // _BN
    have_next = nxt is not None
    nxt_ops = list(nxt) if have_next else []

    def body(h_r, sl_r, sh_r, d_r, mw2_r, mb2_r,
             uw1_r, ub1_r, uw2_r, ub2_r, *rest):
        deg = d_r[0, :, 0:1]
        agg = (_mm(sl_r[0], mw2_r[:LANE, :]) + _mm(sh_r[0], mw2_r[LANE:, :])
               + deg * mb2_r[...])
        hh = h_r[...]
        t = jnp.maximum(_mm(hh, uw1_r[:2 * LANE, :])
                        + _mm(agg, uw1_r[2 * LANE:, :]) + ub1_r[...], 0.0)
        hn = hh + _mm(t, uw2_r[...]) + ub2_r[...]
        if have_next:
            w1a_r, w1b_r, hn_ref, a_ref, b_ref = rest
            a_ref[...] = _split_halves(_mm(hn, w1a_r[...]))
            b_ref[...] = _split_halves(_mm(hn, w1b_r[...]))
        else:
            (hn_ref,) = rest
        hn_ref[...] = hn

    half_spec = [pl.BlockSpec((1, _BN, LANE), lambda i: (0, i, 0)),
                 pl.BlockSpec((1, _BN, LANE), lambda i: (1, i, 0))]
    in_specs = ([pl.BlockSpec((_BN, 2 * LANE), lambda i: (i, 0))]
                + half_spec
                + [pl.BlockSpec((1, _BN, LANE), lambda i: (0, i, 0))]
                + [_full(mw2), _full(mb2), _full(uw1), _full(ub1),
                   _full(uw2), _full(ub2)]
                + [_full(w) for w in nxt_ops])
    out_specs = [pl.BlockSpec((_BN, 2 * LANE), lambda i: (i, 0))]
    out_shape = [jax.ShapeDtypeStruct((n, 2 * LANE), jnp.float32)]
    if have_next:
        out_specs += [pl.BlockSpec((2, _BN, LANE), lambda i: (0, i, 0))] * 2
        out_shape += [jax.ShapeDtypeStruct((2, n_acc, LANE), jnp.float32)] * 2

    outs = pl.pallas_call(
        body,
        grid=(nb,),
        in_specs=in_specs,
        out_specs=out_specs,
        out_shape=out_shape,
    )(h, s2, s2, d2, mw2, mb2, uw1, ub1, uw2, ub2, *nxt_ops)
    return outs if have_next else (outs[0], None, None)


def _tc_head(h, hw1, hb1, hw2, hb2, *, n):
    nb = n // _BN

    def body(h_r, w1_r, b1_r, w2_r, b2_r, o_r):
        o_r[...] = _mm(jnp.maximum(_mm(h_r[...], w1_r[...]) + b1_r[...], 0.0),
                       w2_r[...]) + b2_r[...]

    out = pl.pallas_call(
        body,
        grid=(nb,),
        in_specs=[pl.BlockSpec((_BN, 2 * LANE), lambda i: (i, 0)),
                  _full(hw1), _full(hb1), _full(hw2), _full(hb2)],
        out_specs=pl.BlockSpec((_BN, 1), lambda i: (i, 0)),
        out_shape=jax.ShapeDtypeStruct((n, 1), jnp.float32),
    )(h, hw1, hb1, hw2, hb2)
    return out


def kernel(node_features, edges, edge_features, params):
    p = params
    n, f = node_features.shape
    e, de = edge_features.shape
    h_dim = p['enc_W1'].shape[1]
    assert h_dim == 2 * LANE

    tpt = -(-e // (NS * SB * BE)) * (SB * BE)
    e_pad = tpt * NS
    n_acc = -(-(n + 1) // (NS * 8)) * (NS * 8)

    pad = e_pad - e
    src_p = jnp.concatenate([edges[:, 0], jnp.zeros((pad,), edges.dtype)])
    dst_p = jnp.concatenate([edges[:, 1],
                             jnp.full((pad,), n, edges.dtype)])
    src_p = src_p.astype(jnp.int32)
    dst_p = dst_p.astype(jnp.int32)
    ef_p = jnp.concatenate([edge_features,
                            jnp.zeros((pad, de), jnp.float32)], axis=0)
    srcx2 = jnp.concatenate([src_p, src_p + n_acc])
    dstx2 = jnp.concatenate([dst_p, dst_p + n_acc])

    def rowvec(b):
        return b.reshape(1, -1)

    hh = 2 * LANE
    d2 = None

    h, a2, b2 = _tc_pre(node_features, p['enc_W1'], rowvec(p['enc_b1']),
                        p['enc_W2'], rowvec(p['enc_b2']),
                        p['msg0_W1'][:hh], p['msg0_W1'][hh:2 * hh],
                        n=n, n_acc=n_acc)
    c2_0, c2_1 = _tc_c(ef_p, p['msg0_W1'][2 * hh:], rowvec(p['msg0_b1']),
                       p['msg1_W1'][2 * hh:], rowvec(p['msg1_b1']),
                       e_pad=e_pad)

    rounds = 2
    for r in range(rounds):
        c2 = (c2_0, c2_1)[r]
        s2, d2r = _edge_pass_kernel(n_acc, e_pad)(
            srcx2.reshape(-1, BE), dstx2.reshape(-1, BE),
            dst_p.reshape(-1, BE),
            a2.reshape(NC * n_acc, LANE),
            b2.reshape(NC * n_acc, LANE),
            c2.reshape(NC * e_pad, LANE))
        s2 = s2.reshape(NC, n_acc, LANE)
        if d2 is None:
            d2 = d2r.reshape(NC, n_acc, LANE)
        nxt = None
        if r + 1 < rounds:
            w1n = p[f'msg{r + 1}_W1']
            nxt = (w1n[:hh], w1n[hh:2 * hh])
        h, a2, b2 = _tc_post(h, s2, d2,
                             p[f'msg{r}_W2'], rowvec(p[f'msg{r}_b2']),
                             p[f'upd{r}_W1'], rowvec(p[f'upd{r}_b1']),
                             p[f'upd{r}_W2'], rowvec(p[f'upd{r}_b2']),
                             nxt, n=n, n_acc=n_acc)

    out = _tc_head(h, p['head_W1'], rowvec(p['head_b1']),
                   p['head_W2'], rowvec(p['head_b2']), n=n)
    return out[:, 0]


# speedup vs baseline: 3.4721x; 1.4021x over previous
import functools

import jax
import jax.numpy as jnp
from jax import lax
from jax.experimental import pallas as pl
from jax.experimental.pallas import tpu as pltpu
from jax.experimental.pallas import tpu_sc as plsc

NC = 2
NS = 16
LANE = 16
BE = 128
SB = 4


def _row_mesh():
    return plsc.VectorSubcoreMesh(core_axis_name="c", subcore_axis_name="s")


@functools.lru_cache(maxsize=None)
def _edge_pass_kernel(n_acc, e_pad):
    tpt = e_pad // NS
    sbe = SB * BE
    ng = tpt // sbe
    rpt = tpt // BE
    rows_t = n_acc // NS

    @functools.partial(
        pl.kernel,
        mesh=_row_mesh(),
        out_type=(jax.ShapeDtypeStruct((NC * n_acc, LANE), jnp.float32),
                  jax.ShapeDtypeStruct((NC * n_acc, LANE), jnp.float32)),
        compiler_params=pltpu.CompilerParams(use_tc_tiling_on_sc=False),
        scratch_types=[
            pltpu.VMEM((SB, BE), jnp.int32),
            pltpu.VMEM((SB, BE), jnp.int32),
            pltpu.VMEM((SB, BE), jnp.int32),
            pltpu.VMEM((SB * BE, LANE), jnp.float32),
            pltpu.VMEM((SB * BE, LANE), jnp.float32),
            pltpu.VMEM((SB * BE // 8, 8 * LANE), jnp.float32),
            pltpu.VMEM_SHARED((n_acc, LANE), jnp.float32),
            pltpu.SemaphoreType.DMA,
            pltpu.SemaphoreType.DMA,
            pltpu.SemaphoreType.DMA,
        ],
    )
    def k(srcx2_h, dstx2_h, dst2_h, a2_h, b2_h, c2_h, s_out_h, d_out_h,
          sidx, didx, bidx, arow, brow, cpk, acc, sem_a, sem_b, sem_c):
        c = lax.axis_index("c")
        s = lax.axis_index("s")
        half_off = c * n_acc
        zr = SB * BE

        def fill(val, nrows):
            def fb(i, carry):
                arow[i, :] = jnp.full((LANE,), val, jnp.float32)
                return carry
            lax.fori_loop(0, nrows, fb, 0)

        row0 = s * rows_t
        tile_base = s * tpt
        tile_rbase = s * rpt

        def zero_acc():
            fill(0.0, zr)
            nz = rows_t // zr
            for q in range(nz):
                pltpu.sync_copy(arow.at[pl.ds(0, zr)],
                                acc.at[pl.ds(row0 + q * zr, zr)])
            rem = rows_t - nz * zr
            if rem:
                pltpu.sync_copy(arow.at[pl.ds(0, rem)],
                                acc.at[pl.ds(row0 + nz * zr, rem)])

        zero_acc()
        plsc.subcore_barrier()

        fill(1.0, BE)

        def dbatch(g, carry):
            pltpu.sync_copy(dst2_h.at[pl.ds(tile_rbase + g * SB, SB)], didx)
            for q in range(SB):
                pltpu.sync_copy(arow.at[pl.ds(0, BE)],
                                acc.at[didx.at[q]], add=True)
            return carry
        lax.fori_loop(0, ng, dbatch, 0)
        plsc.subcore_barrier()
        pltpu.sync_copy(acc.at[pl.ds(row0, rows_t)],
                        d_out_h.at[pl.ds(half_off + row0, rows_t)])
        plsc.subcore_barrier()

        zero_acc()
        plsc.subcore_barrier()

        def batch(g, carry):
            ebase = tile_base + g * sbe
            rbase = tile_rbase + g * SB
            cc = pltpu.async_copy(
                c2_h.at[pl.ds((c * e_pad + ebase) // 8, sbe // 8)], cpk, sem_c)
            pltpu.sync_copy(srcx2_h.at[pl.ds(c * rpt * NS + rbase, SB)], sidx)
            pltpu.sync_copy(dstx2_h.at[pl.ds(c * rpt * NS + rbase, SB)], bidx)
            pltpu.sync_copy(dst2_h.at[pl.ds(rbase, SB)], didx)
            gas = [pltpu.async_copy(a2_h.at[sidx.at[q]],
                                    arow.at[pl.ds(q * BE, BE)], sem_a)
                   for q in range(SB)]
            gbs = [pltpu.async_copy(b2_h.at[bidx.at[q]],
                                    brow.at[pl.ds(q * BE, BE)], sem_b)
                   for q in range(SB)]
            for d in gas:
                d.wait()
            for d in gbs:
                d.wait()
            cc.wait()

            def comp(pr, carry2):
                for q in range(8):
                    i = pr * 8 + q
                    arow[i, :] = jnp.maximum(
                        arow[i, :] + brow[i, :]
                        + cpk[pr, pl.ds(q * LANE, LANE)], 0.0)
                return carry2
            lax.fori_loop(0, sbe // 8, comp, 0)
            for q in range(SB):
                pltpu.sync_copy(arow.at[pl.ds(q * BE, BE)],
                                acc.at[didx.at[q]], add=True)
            return carry

        lax.fori_loop(0, ng, batch, 0)
        plsc.subcore_barrier()
        pltpu.sync_copy(acc.at[pl.ds(row0, rows_t)],
                        s_out_h.at[pl.ds(half_off + row0, rows_t)])

    return k


def _mm(a, b):
    return jax.lax.dot_general(a, b, (((1,), (0,)), ((), ())),
                               precision=jax.lax.Precision.HIGHEST)


_BN = 2000


def _full(a):
    return pl.BlockSpec(a.shape, lambda i: tuple(0 for _ in a.shape))


def _split_halves(x):
    return jnp.stack([x[:, :LANE], x[:, LANE:]], axis=0)


def _tc_pre(x, ew1, eb1, ew2, eb2, w1a, w1b, *, n, n_acc):
    nb = n // _BN

    def body(x_r, ew1_r, eb1_r, ew2_r, eb2_r, w1a_r, w1b_r, h_r, a_r, b_r):
        h = _mm(jnp.maximum(_mm(x_r[...], ew1_r[...]) + eb1_r[...], 0.0),
                ew2_r[...]) + eb2_r[...]
        h_r[...] = h
        a_r[...] = _split_halves(_mm(h, w1a_r[...]))
        b_r[...] = _split_halves(_mm(h, w1b_r[...]))

    h, a2, b2 = pl.pallas_call(
        body,
        grid=(nb,),
        in_specs=[pl.BlockSpec((_BN, x.shape[1]), lambda i: (i, 0)),
                  _full(ew1), _full(eb1), _full(ew2), _full(eb2),
                  _full(w1a), _full(w1b)],
        out_specs=[pl.BlockSpec((_BN, 2 * LANE), lambda i: (i, 0)),
                   pl.BlockSpec((2, _BN, LANE), lambda i: (0, i, 0)),
                   pl.BlockSpec((2, _BN, LANE), lambda i: (0, i, 0))],
        out_shape=[jax.ShapeDtypeStruct((n, 2 * LANE), jnp.float32),
                   jax.ShapeDtypeStruct((2, n_acc, LANE), jnp.float32),
                   jax.ShapeDtypeStruct((2, n_acc, LANE), jnp.float32)],
    )(x, ew1, eb1, ew2, eb2, w1a, w1b)
    return h, a2, b2


def _tc_c(efg, wb0, bb0, wb1, bb1, *, e_pad):
    nb = 256
    ebn8 = e_pad // 8 // nb

    def body(e_r, w0_r, b0_r, w1_r, b1_r, c0_r, c1_r):
        e = e_r[...]
        c0_r[...] = jnp.stack([_mm(e, w0_r[0]), _mm(e, w0_r[1])]) + b0_r[...]
        c1_r[...] = jnp.stack([_mm(e, w1_r[0]), _mm(e, w1_r[1])]) + b1_r[...]

    c0, c1 = pl.pallas_call(
        body,
        grid=(nb,),
        in_specs=[pl.BlockSpec((ebn8, efg.shape[1]), lambda i: (i, 0)),
                  _full(wb0), _full(bb0), _full(wb1), _full(bb1)],
        out_specs=[pl.BlockSpec((2, ebn8, 8 * LANE), lambda i: (0, i, 0)),
                   pl.BlockSpec((2, ebn8, 8 * LANE), lambda i: (0, i, 0))],
        out_shape=[jax.ShapeDtypeStruct((2, e_pad // 8, 8 * LANE), jnp.float32),
                   jax.ShapeDtypeStruct((2, e_pad // 8, 8 * LANE), jnp.float32)],
    )(efg, wb0, bb0, wb1, bb1)
    return c0, c1


def _tc_post(h, s2, d2, mw2, mb2, uw1, ub1, uw2, ub2, nxt, *, n, n_acc):
    nb = n // _BN
    have_next = nxt is not None
    nxt_ops = list(nxt) if have_next else []

    def body(h_r, sl_r, sh_r, d_r, mw2_r, mb2_r,
             uw1_r, ub1_r, uw2_r, ub2_r, *rest):
        deg = d_r[0, :, 0:1]
        agg = (_mm(sl_r[0], mw2_r[:LANE, :]) + _mm(sh_r[0], mw2_r[LANE:, :])
               + deg * mb2_r[...])
        hh = h_r[...]
        t = jnp.maximum(_mm(hh, uw1_r[:2 * LANE, :])
                        + _mm(agg, uw1_r[2 * LANE:, :]) + ub1_r[...], 0.0)
        hn = hh + _mm(t, uw2_r[...]) + ub2_r[...]
        if have_next:
            w1a_r, w1b_r, hn_ref, a_ref, b_ref = rest
            a_ref[...] = _split_halves(_mm(hn, w1a_r[...]))
            b_ref[...] = _split_halves(_mm(hn, w1b_r[...]))
        else:
            (hn_ref,) = rest
        hn_ref[...] = hn

    half_spec = [pl.BlockSpec((1, _BN, LANE), lambda i: (0, i, 0)),
                 pl.BlockSpec((1, _BN, LANE), lambda i: (1, i, 0))]
    in_specs = ([pl.BlockSpec((_BN, 2 * LANE), lambda i: (i, 0))]
                + half_spec
                + [pl.BlockSpec((1, _BN, LANE), lambda i: (0, i, 0))]
                + [_full(mw2), _full(mb2), _full(uw1), _full(ub1),
                   _full(uw2), _full(ub2)]
                + [_full(w) for w in nxt_ops])
    out_specs = [pl.BlockSpec((_BN, 2 * LANE), lambda i: (i, 0))]
    out_shape = [jax.ShapeDtypeStruct((n, 2 * LANE), jnp.float32)]
    if have_next:
        out_specs += [pl.BlockSpec((2, _BN, LANE), lambda i: (0, i, 0))] * 2
        out_shape += [jax.ShapeDtypeStruct((2, n_acc, LANE), jnp.float32)] * 2

    outs = pl.pallas_call(
        body,
        grid=(nb,),
        in_specs=in_specs,
        out_specs=out_specs,
        out_shape=out_shape,
    )(h, s2, s2, d2, mw2, mb2, uw1, ub1, uw2, ub2, *nxt_ops)
    return outs if have_next else (outs[0], None, None)


def _tc_head(h, hw1, hb1, hw2, hb2, *, n):
    nb = n // _BN

    def body(h_r, w1_r, b1_r, w2_r, b2_r, o_r):
        o_r[...] = _mm(jnp.maximum(_mm(h_r[...], w1_r[...]) + b1_r[...], 0.0),
                       w2_r[...]) + b2_r[...]

    out = pl.pallas_call(
        body,
        grid=(nb,),
        in_specs=[pl.BlockSpec((_BN, 2 * LANE), lambda i: (i, 0)),
                  _full(hw1), _full(hb1), _full(hw2), _full(hb2)],
        out_specs=pl.BlockSpec((_BN, 1), lambda i: (i, 0)),
        out_shape=jax.ShapeDtypeStruct((n, 1), jnp.float32),
    )(h, hw1, hb1, hw2, hb2)
    return out


def kernel(node_features, edges, edge_features, params):
    p = params
    n, f = node_features.shape
    e, de = edge_features.shape
    h_dim = p['enc_W1'].shape[1]
    assert h_dim == 2 * LANE

    tpt = -(-e // (NS * SB * BE)) * (SB * BE)
    e_pad = tpt * NS
    n_acc = -(-(n + 1) // (NS * 8)) * (NS * 8)

    pad = e_pad - e
    src_p = jnp.concatenate([edges[:, 0], jnp.zeros((pad,), edges.dtype)])
    dst_p = jnp.concatenate([edges[:, 1],
                             jnp.full((pad,), n, edges.dtype)])
    src_p = src_p.astype(jnp.int32)
    dst_p = dst_p.astype(jnp.int32)
    ef_p = jnp.concatenate([edge_features,
                            jnp.zeros((pad, de), jnp.float32)], axis=0)
    srcx2 = jnp.concatenate([src_p, src_p + n_acc])
    dstx2 = jnp.concatenate([dst_p, dst_p + n_acc])

    def rowvec(b):
        return b.reshape(1, -1)

    hh = 2 * LANE
    d2 = None

    h, a2, b2 = _tc_pre(node_features, p['enc_W1'], rowvec(p['enc_b1']),
                        p['enc_W2'], rowvec(p['enc_b2']),
                        p['msg0_W1'][:hh], p['msg0_W1'][hh:2 * hh],
                        n=n, n_acc=n_acc)
    def blkw(w1):
        wc = w1[2 * hh:]
        return jnp.stack([jnp.kron(jnp.eye(8, dtype=jnp.float32), wc[:, :LANE]),
                          jnp.kron(jnp.eye(8, dtype=jnp.float32), wc[:, LANE:])])

    def blkb(b1):
        return jnp.stack([jnp.tile(b1[:LANE], 8),
                          jnp.tile(b1[LANE:], 8)]).reshape(2, 1, 8 * LANE)

    c2_0, c2_1 = _tc_c(ef_p.reshape(-1, 8 * de),
                       blkw(p['msg0_W1']), blkb(p['msg0_b1']),
                       blkw(p['msg1_W1']), blkb(p['msg1_b1']),
                       e_pad=e_pad)

    rounds = 2
    for r in range(rounds):
        c2 = (c2_0, c2_1)[r]
        s2, d2r = _edge_pass_kernel(n_acc, e_pad)(
            srcx2.reshape(-1, BE), dstx2.reshape(-1, BE),
            dst_p.reshape(-1, BE),
            a2.reshape(NC * n_acc, LANE),
            b2.reshape(NC * n_acc, LANE),
            c2.reshape(NC * e_pad // 8, 8 * LANE))
        s2 = s2.reshape(NC, n_acc, LANE)
        if d2 is None:
            d2 = d2r.reshape(NC, n_acc, LANE)
        nxt = None
        if r + 1 < rounds:
            w1n = p[f'msg{r + 1}_W1']
            nxt = (w1n[:hh], w1n[hh:2 * hh])
        h, a2, b2 = _tc_post(h, s2, d2,
                             p[f'msg{r}_W2'], rowvec(p[f'msg{r}_b2']),
                             p[f'upd{r}_W1'], rowvec(p[f'upd{r}_b1']),
                             p[f'upd{r}_W2'], rowvec(p[f'upd{r}_b2']),
                             nxt, n=n, n_acc=n_acc)

    out = _tc_head(h, p['head_W1'], rowvec(p['head_b1']),
                   p['head_W2'], rowvec(p['head_b2']), n=n)
    return out[:, 0]
